# baseline (speedup 1.0000x reference)
import functools
import math

import jax
import jax.numpy as jnp
from jax.experimental import pallas as pl
from jax.experimental.pallas import tpu as pltpu
from jax.experimental.pallas import tpu_sc as plsc

_BATCH = 16
_SEQ = 24
_N = 512
_D = 128
_FEAT = 8
_TAU = 3
_K = 8
_BIG = 3.0e38


def _dotT(a, b):
    return jax.lax.dot_general(a, b, (((1,), (1,)), ((), ())),
                               preferred_element_type=jnp.float32)


def _dot(a, b):
    return jax.lax.dot_general(a, b, (((1,), (0,)), ((), ())),
                               preferred_element_type=jnp.float32)


def _lane_bcast(v, jj):
    dnums = jax.lax.GatherDimensionNumbers(
        offset_dims=(), collapsed_slice_dims=(0,), start_index_map=(0,))
    idx = jnp.full((16, 1), jj, jnp.int32)
    return jax.lax.gather(v, idx, dnums, (1,),
                          mode=jax.lax.GatherScatterMode.PROMISE_IN_BOUNDS)


_NC = 2
_NS = 16
_ROWS_PER_TILE = _BATCH * _N // (_NC * _NS)


@functools.partial(
    pl.kernel,
    out_type=jax.ShapeDtypeStruct((_BATCH * _N,), jnp.float32),
    mesh=plsc.VectorSubcoreMesh(core_axis_name="c", subcore_axis_name="s"),
    scratch_types=[
        pltpu.VMEM((_ROWS_PER_TILE,), jnp.float32),
        pltpu.VMEM((_ROWS_PER_TILE,), jnp.float32),
        pltpu.VMEM((_N,), jnp.float32),
        pltpu.VMEM((_N,), jnp.float32),
        pltpu.VMEM((_ROWS_PER_TILE,), jnp.float32),
    ],
)
def _sc_knn(wx_hbm, wy_hbm, out_hbm,
            wxr_v, wyr_v, wxc_v, wyc_v, tau_v):
    c = jax.lax.axis_index("c")
    s = jax.lax.axis_index("s")
    wid = s * _NC + c
    base = wid * _ROWS_PER_TILE
    bbase = (wid // _NC) * _N
    pltpu.sync_copy(wx_hbm.at[pl.ds(base, _ROWS_PER_TILE)], wxr_v)
    pltpu.sync_copy(wy_hbm.at[pl.ds(base, _ROWS_PER_TILE)], wyr_v)
    pltpu.sync_copy(wx_hbm.at[pl.ds(bbase, _N)], wxc_v)
    pltpu.sync_copy(wy_hbm.at[pl.ds(bbase, _N)], wyc_v)

    def group(g, _):
        wxg = wxr_v[pl.ds(g * 16, 16)]
        wyg = wyr_v[pl.ds(g * 16, 16)]

        def chunk(ci, r):
            cvx = wxc_v[pl.ds(ci * 16, 16)]
            cvy = wyc_v[pl.ds(ci * 16, 16)]
            for jj in range(16):
                dx = wxg - _lane_bcast(cvx, jj)
                dy = wyg - _lane_bcast(cvy, jj)
                hi = dx * dx + dy * dy
                out = []
                for i in range(_K):
                    lo = jnp.minimum(r[i], hi)
                    hi = jnp.maximum(r[i], hi)
                    out.append(lo)
                r = tuple(out)
            return r

        init = tuple(jnp.full((16,), _BIG, jnp.float32) for _ in range(_K))
        r = jax.lax.fori_loop(0, _N // 16, chunk, init)
        tau_v[pl.ds(g * 16, 16)] = r[_K - 1]
        return 0

    jax.lax.fori_loop(0, _ROWS_PER_TILE // 16, group, 0)
    pltpu.sync_copy(tau_v, out_hbm.at[pl.ds(base, _ROWS_PER_TILE)])


def _body(h0_ref, h1_ref, h2_ref, wcol_ref, wrow_ref,
          wq_ref, wk_ref, wv_ref, w1_ref, w2_ref,
          bq_ref, bv_ref, b1_ref, b2_ref,
          out_ref):
    wc = wcol_ref[0]
    wr = wrow_ref[0]
    wxc = jax.lax.broadcast_in_dim(wc[:, 0:1], (_N, _N), (0, 1))
    wyc = jax.lax.broadcast_in_dim(wc[:, 1:2], (_N, _N), (0, 1))
    wxr = jax.lax.broadcast_in_dim(wr[0:1, :], (_N, _N), (0, 1))
    wyr = jax.lax.broadcast_in_dim(wr[1:2, :], (_N, _N), (0, 1))
    dx = wxc - wxr
    dy = wyc - wyr
    d2 = dx * dx + dy * dy

    tauc = jax.lax.broadcast_in_dim(wc[:, 2:3], (_N, _N), (0, 1))
    mask = d2 <= tauc

    he_last = h2_ref[0, 0]
    q = _dotT(he_last, wq_ref[...]) + bq_ref[...]
    qk = _dot(q, wk_ref[...]).astype(jnp.bfloat16)

    inv = 1.0 / math.sqrt(_D)
    den = jnp.zeros((_N, 1), jnp.float32)
    ctx = jnp.zeros((_N, _D), jnp.float32)
    for h_ref in (h0_ref, h1_ref, h2_ref):
        h_t = h_ref[0, 0].astype(jnp.bfloat16)
        s_t = _dotT(qk, h_t)
        e_t = jnp.where(mask, jnp.exp(s_t * inv), 0.0)
        den = den + jnp.sum(e_t, axis=1, keepdims=True)
        ctx = ctx + _dot(e_t.astype(jnp.bfloat16), h_t)
    ctx = ctx / den

    ctx = _dotT(ctx, wv_ref[...]) + bv_ref[...]
    h1 = _dotT(ctx, w1_ref[...]) + b1_ref[...]
    g = 0.5 * h1 * (1.0 + jnp.tanh(0.7978845608028654 *
                                   (h1 + 0.044715 * h1 * h1 * h1)))
    out_ref[0] = _dotT(g, w2_ref[...]) + b2_ref[...]


@jax.jit
def kernel(h_e, x_orig, Wq, bq, Wk, bk, Wv, bv, W1, b1, W2, b2):
    b, seq_len, n, d = h_e.shape
    t0 = seq_len - 1
    t_start = t0 - _TAU + 1

    last_wind = x_orig[t0, :, :, 4:6]
    tau = _sc_knn(last_wind[:, :, 0].reshape(-1), last_wind[:, :, 1].reshape(-1))
    wcol = jnp.concatenate(
        [last_wind, tau.reshape(b, n, 1),
         jnp.zeros((b, n, _FEAT - 3), jnp.float32)], axis=-1)
    wrow = jnp.pad(jnp.transpose(last_wind, (0, 2, 1)), ((0, 0), (0, 6), (0, 0)))

    full = lambda shape: pl.BlockSpec(shape, lambda i: (0,) * len(shape))
    h_t_spec = lambda t: pl.BlockSpec((1, 1, n, d), lambda i, t=t: (i, t, 0, 0))

    out = pl.pallas_call(
        _body,
        grid=(b,),
        in_specs=[
            h_t_spec(t_start), h_t_spec(t_start + 1), h_t_spec(t0),
            pl.BlockSpec((1, n, _FEAT), lambda i: (i, 0, 0)),
            pl.BlockSpec((1, 8, n), lambda i: (i, 0, 0)),
            full((d, d)), full((d, d)), full((d, d)), full((d, d)), full((d, d)),
            full((1, d)), full((1, d)), full((1, d)), full((1, d)),
        ],
        out_specs=pl.BlockSpec((1, n, d), lambda i: (i, 0, 0)),
        out_shape=jax.ShapeDtypeStruct((b, n, d), jnp.float32),
    )(h_e, h_e, h_e, wcol, wrow, Wq, Wk, Wv, W1, W2,
      bq.reshape(1, d), bv.reshape(1, d), b1.reshape(1, d), b2.reshape(1, d))
    return out


# speedup vs baseline: 1.0031x; 1.0031x over previous
import functools
import math

import jax
import jax.numpy as jnp
from jax.experimental import pallas as pl
from jax.experimental.pallas import tpu as pltpu
from jax.experimental.pallas import tpu_sc as plsc

_BATCH = 16
_SEQ = 24
_N = 512
_D = 128
_FEAT = 8
_TAU = 3
_K = 8
_BIG = 3.0e38


def _dotT(a, b):
    return jax.lax.dot_general(a, b, (((1,), (1,)), ((), ())),
                               preferred_element_type=jnp.float32)


def _dot(a, b):
    return jax.lax.dot_general(a, b, (((1,), (0,)), ((), ())),
                               preferred_element_type=jnp.float32)


def _lane_bcast(v, jj):
    dnums = jax.lax.GatherDimensionNumbers(
        offset_dims=(), collapsed_slice_dims=(0,), start_index_map=(0,))
    idx = jnp.full((16, 1), jj, jnp.int32)
    return jax.lax.gather(v, idx, dnums, (1,),
                          mode=jax.lax.GatherScatterMode.PROMISE_IN_BOUNDS)


_NC = 2
_NS = 16
_ROWS_PER_TILE = _BATCH * _N // (_NC * _NS)


@functools.partial(
    pl.kernel,
    out_type=jax.ShapeDtypeStruct((_BATCH * _N,), jnp.float32),
    mesh=plsc.VectorSubcoreMesh(core_axis_name="c", subcore_axis_name="s",
                                num_cores=_NC),
    scratch_types=[
        pltpu.VMEM((_ROWS_PER_TILE,), jnp.float32),
        pltpu.VMEM((_ROWS_PER_TILE,), jnp.float32),
        pltpu.VMEM((_N,), jnp.float32),
        pltpu.VMEM((_N,), jnp.float32),
        pltpu.VMEM((_ROWS_PER_TILE,), jnp.float32),
    ],
)
def _sc_knn(wx_hbm, wy_hbm, out_hbm,
            wxr_v, wyr_v, wxc_v, wyc_v, tau_v):
    c = jax.lax.axis_index("c")
    s = jax.lax.axis_index("s")
    wid = s * _NC + c
    base = wid * _ROWS_PER_TILE
    bbase = (wid // _NC) * _N
    pltpu.sync_copy(wx_hbm.at[pl.ds(base, _ROWS_PER_TILE)], wxr_v)
    pltpu.sync_copy(wy_hbm.at[pl.ds(base, _ROWS_PER_TILE)], wyr_v)
    pltpu.sync_copy(wx_hbm.at[pl.ds(bbase, _N)], wxc_v)
    pltpu.sync_copy(wy_hbm.at[pl.ds(bbase, _N)], wyc_v)

    def group(g, _):
        wxg = wxr_v[pl.ds(g * 16, 16)]
        wyg = wyr_v[pl.ds(g * 16, 16)]

        def chunk(ci, r):
            cvx = wxc_v[pl.ds(ci * 16, 16)]
            cvy = wyc_v[pl.ds(ci * 16, 16)]
            for jj in range(16):
                dx = wxg - _lane_bcast(cvx, jj)
                dy = wyg - _lane_bcast(cvy, jj)
                hi = dx * dx + dy * dy
                out = []
                for i in range(_K):
                    lo = jnp.minimum(r[i], hi)
                    hi = jnp.maximum(r[i], hi)
                    out.append(lo)
                r = tuple(out)
            return r

        init = tuple(jnp.full((16,), _BIG, jnp.float32) for _ in range(_K))
        r = jax.lax.fori_loop(0, _N // 16, chunk, init)
        tau_v[pl.ds(g * 16, 16)] = r[_K - 1]
        return 0

    jax.lax.fori_loop(0, _ROWS_PER_TILE // 16, group, 0)
    pltpu.sync_copy(tau_v, out_hbm.at[pl.ds(base, _ROWS_PER_TILE)])


def _body(h0_ref, h1_ref, h2_ref, wcol_ref, wrow_ref,
          wq_ref, wk_ref, wv_ref, w1_ref, w2_ref,
          bq_ref, bv_ref, b1_ref, b2_ref,
          out_ref):
    wc = wcol_ref[0]
    wr = wrow_ref[0]
    wxc = jax.lax.broadcast_in_dim(wc[:, 0:1], (_N, _N), (0, 1))
    wyc = jax.lax.broadcast_in_dim(wc[:, 1:2], (_N, _N), (0, 1))
    wxr = jax.lax.broadcast_in_dim(wr[0:1, :], (_N, _N), (0, 1))
    wyr = jax.lax.broadcast_in_dim(wr[1:2, :], (_N, _N), (0, 1))
    dx = wxc - wxr
    dy = wyc - wyr
    d2 = dx * dx + dy * dy

    tauc = jax.lax.broadcast_in_dim(wc[:, 2:3], (_N, _N), (0, 1))
    mask = d2 <= tauc

    he_last = h2_ref[0, 0]
    q = _dotT(he_last, wq_ref[...]) + bq_ref[...]
    qk = _dot(q, wk_ref[...]).astype(jnp.bfloat16)

    inv = 1.0 / math.sqrt(_D)
    den = jnp.zeros((_N, 1), jnp.float32)
    ctx = jnp.zeros((_N, _D), jnp.float32)
    for h_ref in (h0_ref, h1_ref, h2_ref):
        h_t = h_ref[0, 0].astype(jnp.bfloat16)
        s_t = _dotT(qk, h_t)
        e_t = jnp.where(mask, jnp.exp(s_t * inv), 0.0)
        den = den + jnp.sum(e_t, axis=1, keepdims=True)
        ctx = ctx + _dot(e_t.astype(jnp.bfloat16), h_t)
    ctx = ctx / den

    ctx = _dotT(ctx, wv_ref[...]) + bv_ref[...]
    h1 = _dotT(ctx, w1_ref[...]) + b1_ref[...]
    g = 0.5 * h1 * (1.0 + jnp.tanh(0.7978845608028654 *
                                   (h1 + 0.044715 * h1 * h1 * h1)))
    out_ref[0] = _dotT(g, w2_ref[...]) + b2_ref[...]


@jax.jit
def kernel(h_e, x_orig, Wq, bq, Wk, bk, Wv, bv, W1, b1, W2, b2):
    b, seq_len, n, d = h_e.shape
    t0 = seq_len - 1
    t_start = t0 - _TAU + 1

    last_wind = x_orig[t0, :, :, 4:6]
    tau = _sc_knn(last_wind[:, :, 0].reshape(-1), last_wind[:, :, 1].reshape(-1))
    wcol = jnp.concatenate(
        [last_wind, tau.reshape(b, n, 1),
         jnp.zeros((b, n, _FEAT - 3), jnp.float32)], axis=-1)
    wrow = jnp.pad(jnp.transpose(last_wind, (0, 2, 1)), ((0, 0), (0, 6), (0, 0)))

    full = lambda shape: pl.BlockSpec(shape, lambda i: (0,) * len(shape))
    h_t_spec = lambda t: pl.BlockSpec((1, 1, n, d), lambda i, t=t: (i, t, 0, 0))

    out = pl.pallas_call(
        _body,
        grid=(b,),
        in_specs=[
            h_t_spec(t_start), h_t_spec(t_start + 1), h_t_spec(t0),
            pl.BlockSpec((1, n, _FEAT), lambda i: (i, 0, 0)),
            pl.BlockSpec((1, 8, n), lambda i: (i, 0, 0)),
            full((d, d)), full((d, d)), full((d, d)), full((d, d)), full((d, d)),
            full((1, d)), full((1, d)), full((1, d)), full((1, d)),
        ],
        out_specs=pl.BlockSpec((1, n, d), lambda i: (i, 0, 0)),
        out_shape=jax.ShapeDtypeStruct((b, n, d), jnp.float32),
    )(h_e, h_e, h_e, wcol, wrow, Wq, Wk, Wv, W1, W2,
      bq.reshape(1, d), bv.reshape(1, d), b1.reshape(1, d), b2.reshape(1, d))
    return out


# speedup vs baseline: 1.7641x; 1.7587x over previous
import math

import jax
import jax.numpy as jnp
from jax.experimental import pallas as pl
from jax.experimental.pallas import tpu as pltpu

_BATCH = 16
_SEQ = 24
_N = 512
_D = 128
_FEAT = 8
_TAU = 3
_K = 8
_BIG = 3.0e38


def _dotT(a, b):
    return jax.lax.dot_general(a, b, (((1,), (1,)), ((), ())),
                               preferred_element_type=jnp.float32)


def _dot(a, b):
    return jax.lax.dot_general(a, b, (((1,), (0,)), ((), ())),
                               preferred_element_type=jnp.float32)


def _body(h0_ref, h1_ref, h2_ref, wcol_ref, wrow_ref,
          wq_ref, wk_ref, wv_ref, w1_ref, w2_ref,
          bq_ref, bv_ref, b1_ref, b2_ref,
          out_ref):
    wc = wcol_ref[0]
    wr = wrow_ref[0]
    wxc = jax.lax.broadcast_in_dim(wc[:, 0:1], (_N, _N), (0, 1))
    wyc = jax.lax.broadcast_in_dim(wc[:, 1:2], (_N, _N), (0, 1))
    wxr = jax.lax.broadcast_in_dim(wr[0:1, :], (_N, _N), (0, 1))
    wyr = jax.lax.broadcast_in_dim(wr[1:2, :], (_N, _N), (0, 1))
    dx = wxc - wxr
    dy = wyc - wyr
    d2 = dx * dx + dy * dy

    cur = d2
    for _ in range(_K):
        rmin = jnp.min(cur, axis=1, keepdims=True)
        cur = jnp.where(cur == rmin, _BIG, cur)
    mask = cur > d2

    he_last = h2_ref[0, 0]
    q = _dotT(he_last, wq_ref[...]) + bq_ref[...]
    qk = _dot(q, wk_ref[...]).astype(jnp.bfloat16)

    inv = 1.0 / math.sqrt(_D)
    den = jnp.zeros((_N, 1), jnp.float32)
    ctx = jnp.zeros((_N, _D), jnp.float32)
    for h_ref in (h0_ref, h1_ref, h2_ref):
        h_t = h_ref[0, 0].astype(jnp.bfloat16)
        s_t = _dotT(qk, h_t)
        e_t = jnp.where(mask, jnp.exp(s_t * inv), 0.0)
        den = den + jnp.sum(e_t, axis=1, keepdims=True)
        ctx = ctx + _dot(e_t.astype(jnp.bfloat16), h_t)
    ctx = ctx / den

    ctx = _dotT(ctx, wv_ref[...]) + bv_ref[...]
    h1 = _dotT(ctx, w1_ref[...]) + b1_ref[...]
    g = 0.5 * h1 * (1.0 + jnp.tanh(0.7978845608028654 *
                                   (h1 + 0.044715 * h1 * h1 * h1)))
    out_ref[0] = _dotT(g, w2_ref[...]) + b2_ref[...]


@jax.jit
def kernel(h_e, x_orig, Wq, bq, Wk, bk, Wv, bv, W1, b1, W2, b2):
    b, seq_len, n, d = h_e.shape
    t0 = seq_len - 1
    t_start = t0 - _TAU + 1

    last_wind = x_orig[t0, :, :, 4:6]
    wcol = jnp.pad(last_wind, ((0, 0), (0, 0), (0, _FEAT - 2)))
    wrow = jnp.pad(jnp.transpose(last_wind, (0, 2, 1)), ((0, 0), (0, 6), (0, 0)))

    full = lambda shape: pl.BlockSpec(shape, lambda i: (0,) * len(shape))
    h_t_spec = lambda t: pl.BlockSpec((1, 1, n, d), lambda i, t=t: (i, t, 0, 0))

    out = pl.pallas_call(
        _body,
        grid=(b,),
        in_specs=[
            h_t_spec(t_start), h_t_spec(t_start + 1), h_t_spec(t0),
            pl.BlockSpec((1, n, _FEAT), lambda i: (i, 0, 0)),
            pl.BlockSpec((1, 8, n), lambda i: (i, 0, 0)),
            full((d, d)), full((d, d)), full((d, d)), full((d, d)), full((d, d)),
            full((1, d)), full((1, d)), full((1, d)), full((1, d)),
        ],
        out_specs=pl.BlockSpec((1, n, d), lambda i: (i, 0, 0)),
        out_shape=jax.ShapeDtypeStruct((b, n, d), jnp.float32),
        compiler_params=pltpu.CompilerParams(
            dimension_semantics=("parallel",)),
    )(h_e, h_e, h_e, wcol, wrow, Wq, Wk, Wv, W1, W2,
      bq.reshape(1, d), bv.reshape(1, d), b1.reshape(1, d), b2.reshape(1, d))
    return out
